# baseline (speedup 1.0000x reference)
import jax
import jax.numpy as jnp
from jax import lax
from jax.experimental import pallas as pl
from jax.experimental.pallas import tpu as pltpu
from jax.experimental.pallas import tpu_sc as plsc

_NUM_BUCKETS = 1000000
_EMBED_DIM = 16
_BATCH = 16384
_NC = 2
_NS = 16
_L = 16
_NW = _NC * _NS
_BPW = _BATCH // _NW
_CH = 128
_NCH = _BPW // _CH
_GROUPS = _NUM_BUCKETS // 8

_DT_C = 8192
_DT_R = _DT_C // 8
_DT_STEPS = -(-_NUM_BUCKETS // _DT_C)


def _detile_body(x_ref, y_ref, z_ref):
    x = x_ref[...]
    eye = jnp.eye(_EMBED_DIM, dtype=jnp.float32)
    xt = jax.lax.dot_general(x, eye, (((0,), (0,)), ((), ())),
                             preferred_element_type=jnp.float32)
    z_ref[...] = xt.reshape(_DT_R, 8, _EMBED_DIM)
    for s in range(8):
        y_ref[:, pl.ds(s * _EMBED_DIM, _EMBED_DIM)] = z_ref[:, s, :]


def _detile(t2):
    return pl.pallas_call(
        _detile_body,
        grid=(_DT_STEPS,),
        in_specs=[pl.BlockSpec((_EMBED_DIM, _DT_C), lambda i: (0, i))],
        out_specs=pl.BlockSpec((_DT_R, 128), lambda i: (i, 0)),
        out_shape=jax.ShapeDtypeStruct((_GROUPS, 128), jnp.float32),
        scratch_shapes=[pltpu.VMEM((_DT_R, 8, _EMBED_DIM), jnp.float32)],
    )(t2)


def _make_sc_kernel():
    mesh = plsc.VectorSubcoreMesh(core_axis_name="c", subcore_axis_name="s")

    @pl.kernel(
        mesh=mesh,
        out_type=jax.ShapeDtypeStruct((_BATCH * _EMBED_DIM,), jnp.float32),
        scratch_types=[
            pltpu.VMEM((_BPW,), jnp.int32),
            pltpu.VMEM((_NCH, _CH), jnp.int32),
            pltpu.VMEM((_NCH, _CH), jnp.int32),
            pltpu.VMEM((_BPW,), jnp.int32),
            pltpu.VMEM((_BPW,), jnp.int32),
            pltpu.VMEM((_CH, 128), jnp.float32),
            pltpu.VMEM((_CH, 128), jnp.float32),
            pltpu.VMEM((_CH, 128), jnp.float32),
            pltpu.VMEM((_CH, 128), jnp.float32),
            pltpu.VMEM((_BPW * _EMBED_DIM,), jnp.float32),
            pltpu.SemaphoreType.DMA,
            pltpu.SemaphoreType.DMA,
            pltpu.SemaphoreType.DMA,
            pltpu.SemaphoreType.DMA,
        ],
    )
    def dh_embed(feat_hbm, tbl_hbm, out_hbm,
                 feat_v, g1_v, g2_v, s1_v, s2_v,
                 rows1a_v, rows1b_v, rows2a_v, rows2b_v, out_v,
                 sem10, sem11, sem20, sem21):
        wid = lax.axis_index("s") * _NC + lax.axis_index("c")
        base = wid * _BPW
        pltpu.sync_copy(feat_hbm.at[pl.ds(base, _BPW)], feat_v)

        for c in range(_NCH):
            def hash_body(j, _, c=c):
                g = c * (_CH // _L) + j
                x = feat_v[pl.ds(g * _L, _L)].astype(jnp.uint32)
                h1 = (x * jnp.uint32(2654435761)) % jnp.uint32(_NUM_BUCKETS)
                y = x ^ (x >> jnp.uint32(16))
                y = y * jnp.uint32(0x45D9F3B)
                y = y ^ (y >> jnp.uint32(13))
                h2 = y % jnp.uint32(_NUM_BUCKETS)
                h1 = h1.astype(jnp.int32)
                h2 = h2.astype(jnp.int32)
                g1_v[c, pl.ds(j * _L, _L)] = h1 >> 3
                g2_v[c, pl.ds(j * _L, _L)] = h2 >> 3
                s1_v[pl.ds(g * _L, _L)] = (h1 & 7) << 4
                s2_v[pl.ds(g * _L, _L)] = (h2 & 7) << 4
                return 0

            lax.fori_loop(0, _CH // _L, hash_body, 0)

        bufs = ((rows1a_v, rows2a_v, sem10, sem20),
                (rows1b_v, rows2b_v, sem11, sem21))

        def start(c):
            b1, b2, s1, s2 = bufs[c % 2]
            return (
                pltpu.async_copy(tbl_hbm.at[g1_v.at[c]], b1, s1),
                pltpu.async_copy(tbl_hbm.at[g2_v.at[c]], b2, s2),
            )

        pend = start(0)
        for c in range(_NCH):
            nxt = start(c + 1) if c + 1 < _NCH else None
            pend[0].wait()
            pend[1].wait()
            r1, r2 = bufs[c % 2][0], bufs[c % 2][1]

            def extract(j, _, c=c, r1=r1, r2=r2):
                sv1 = s1_v[pl.ds((c * (_CH // _L) + j) * _L, _L)]
                sv2 = s2_v[pl.ds((c * (_CH // _L) + j) * _L, _L)]
                for l in range(_L):
                    i = j * _L + l
                    v1 = r1[i, pl.ds(sv1[l], _L)]
                    v2 = r2[i, pl.ds(sv2[l], _L)]
                    out_v[pl.ds((c * _CH + i) * _EMBED_DIM, _L)] = v1 + v2
                return 0

            lax.fori_loop(0, _CH // _L, extract, 0)
            pend = nxt

        pltpu.sync_copy(out_v, out_hbm.at[pl.ds(base * _EMBED_DIM,
                                                _BPW * _EMBED_DIM)])

    return dh_embed


_dh_embed = _make_sc_kernel()


def kernel(features, table):
    tbl_lin = _detile(table.T)
    out = _dh_embed(features, tbl_lin)
    return out.reshape(_BATCH, _EMBED_DIM)


# speedup vs baseline: 1.5166x; 1.5166x over previous
import jax
import jax.numpy as jnp
from jax import lax
from jax.experimental import pallas as pl
from jax.experimental.pallas import tpu as pltpu
from jax.experimental.pallas import tpu_sc as plsc

_NUM_BUCKETS = 1000000
_EMBED_DIM = 16
_BATCH = 16384
_NC = 2
_NS = 16
_L = 16
_NW = _NC * _NS
_BPW = _BATCH // _NW
_CH = 128
_NCH = _BPW // _CH
_GROUPS = _NUM_BUCKETS // 8

_DT_C = 8192
_DT_R = _DT_C // 8
_DT_STEPS = -(-_NUM_BUCKETS // _DT_C)


def _detile_body(x_ref, y_ref):
    x = x_ref[...]
    xt = x.T.reshape(_DT_R, 8, _EMBED_DIM)
    for s in range(8):
        y_ref[:, pl.ds(s * _EMBED_DIM, _EMBED_DIM)] = xt[:, s, :]


def _detile(t2):
    return pl.pallas_call(
        _detile_body,
        grid=(_DT_STEPS,),
        in_specs=[pl.BlockSpec((_EMBED_DIM, _DT_C), lambda i: (0, i))],
        out_specs=pl.BlockSpec((_DT_R, 128), lambda i: (i, 0)),
        out_shape=jax.ShapeDtypeStruct((_GROUPS, 128), jnp.float32),
    )(t2)


def _make_sc_kernel():
    mesh = plsc.VectorSubcoreMesh(core_axis_name="c", subcore_axis_name="s")

    @pl.kernel(
        mesh=mesh,
        out_type=jax.ShapeDtypeStruct((_BATCH * _EMBED_DIM,), jnp.float32),
        scratch_types=[
            pltpu.VMEM((_BPW,), jnp.int32),
            pltpu.VMEM((_NCH, _CH), jnp.int32),
            pltpu.VMEM((_NCH, _CH), jnp.int32),
            pltpu.VMEM((_BPW,), jnp.int32),
            pltpu.VMEM((_BPW,), jnp.int32),
            pltpu.VMEM((_CH, 128), jnp.float32),
            pltpu.VMEM((_CH, 128), jnp.float32),
            pltpu.VMEM((_CH, 128), jnp.float32),
            pltpu.VMEM((_CH, 128), jnp.float32),
            pltpu.VMEM((_BPW * _EMBED_DIM,), jnp.float32),
            pltpu.SemaphoreType.DMA,
            pltpu.SemaphoreType.DMA,
            pltpu.SemaphoreType.DMA,
            pltpu.SemaphoreType.DMA,
        ],
    )
    def dh_embed(feat_hbm, tbl_hbm, out_hbm,
                 feat_v, g1_v, g2_v, s1_v, s2_v,
                 rows1a_v, rows1b_v, rows2a_v, rows2b_v, out_v,
                 sem10, sem11, sem20, sem21):
        wid = lax.axis_index("s") * _NC + lax.axis_index("c")
        base = wid * _BPW
        pltpu.sync_copy(feat_hbm.at[pl.ds(base, _BPW)], feat_v)

        for c in range(_NCH):
            def hash_body(j, _, c=c):
                g = c * (_CH // _L) + j
                x = feat_v[pl.ds(g * _L, _L)].astype(jnp.uint32)
                h1 = (x * jnp.uint32(2654435761)) % jnp.uint32(_NUM_BUCKETS)
                y = x ^ (x >> jnp.uint32(16))
                y = y * jnp.uint32(0x45D9F3B)
                y = y ^ (y >> jnp.uint32(13))
                h2 = y % jnp.uint32(_NUM_BUCKETS)
                h1 = h1.astype(jnp.int32)
                h2 = h2.astype(jnp.int32)
                g1_v[c, pl.ds(j * _L, _L)] = h1 >> 3
                g2_v[c, pl.ds(j * _L, _L)] = h2 >> 3
                s1_v[pl.ds(g * _L, _L)] = (h1 & 7) << 4
                s2_v[pl.ds(g * _L, _L)] = (h2 & 7) << 4
                return 0

            lax.fori_loop(0, _CH // _L, hash_body, 0)

        bufs = ((rows1a_v, rows2a_v, sem10, sem20),
                (rows1b_v, rows2b_v, sem11, sem21))

        def start(c):
            b1, b2, s1, s2 = bufs[c % 2]
            return (
                pltpu.async_copy(tbl_hbm.at[g1_v.at[c]], b1, s1),
                pltpu.async_copy(tbl_hbm.at[g2_v.at[c]], b2, s2),
            )

        pend = start(0)
        for c in range(_NCH):
            nxt = start(c + 1) if c + 1 < _NCH else None
            pend[0].wait()
            pend[1].wait()
            r1, r2 = bufs[c % 2][0], bufs[c % 2][1]

            def extract(j, _, c=c, r1=r1, r2=r2):
                sv1 = s1_v[pl.ds((c * (_CH // _L) + j) * _L, _L)]
                sv2 = s2_v[pl.ds((c * (_CH // _L) + j) * _L, _L)]
                for l in range(_L):
                    i = j * _L + l
                    v1 = r1[i, pl.ds(sv1[l], _L)]
                    v2 = r2[i, pl.ds(sv2[l], _L)]
                    out_v[pl.ds((c * _CH + i) * _EMBED_DIM, _L)] = v1 + v2
                return 0

            lax.fori_loop(0, _CH // _L, extract, 0)
            pend = nxt

        pltpu.sync_copy(out_v, out_hbm.at[pl.ds(base * _EMBED_DIM,
                                                _BPW * _EMBED_DIM)])

    return dh_embed


_dh_embed = _make_sc_kernel()


def kernel(features, table):
    tbl_lin = _detile(table.T)
    out = _dh_embed(features, tbl_lin)
    return out.reshape(_BATCH, _EMBED_DIM)


# speedup vs baseline: 1.5564x; 1.0263x over previous
import jax
import jax.numpy as jnp
from jax import lax
from jax.experimental import pallas as pl
from jax.experimental.pallas import tpu as pltpu
from jax.experimental.pallas import tpu_sc as plsc

_NUM_BUCKETS = 1000000
_EMBED_DIM = 16
_BATCH = 16384
_NC = 2
_NS = 16
_L = 16
_NW = _NC * _NS
_BPW = _BATCH // _NW
_CH = 128
_NCH = _BPW // _CH
_GROUPS = _NUM_BUCKETS // 8

_DT_C = 32768
_DT_R = _DT_C // 8
_DT_STEPS = -(-_NUM_BUCKETS // _DT_C)


def _detile_body(x_ref, y_ref):
    x = x_ref[...]
    xt = x.T.reshape(_DT_R, 8, _EMBED_DIM)
    for s in range(8):
        y_ref[:, pl.ds(s * _EMBED_DIM, _EMBED_DIM)] = xt[:, s, :]


def _detile(t2):
    return pl.pallas_call(
        _detile_body,
        grid=(_DT_STEPS,),
        in_specs=[pl.BlockSpec((_EMBED_DIM, _DT_C), lambda i: (0, i))],
        out_specs=pl.BlockSpec((_DT_R, 128), lambda i: (i, 0)),
        out_shape=jax.ShapeDtypeStruct((_GROUPS, 128), jnp.float32),
    )(t2)


def _make_sc_kernel():
    mesh = plsc.VectorSubcoreMesh(core_axis_name="c", subcore_axis_name="s")

    @pl.kernel(
        mesh=mesh,
        out_type=jax.ShapeDtypeStruct((_BATCH * _EMBED_DIM,), jnp.float32),
        scratch_types=[
            pltpu.VMEM((_BPW,), jnp.int32),
            pltpu.VMEM((_NCH, _CH), jnp.int32),
            pltpu.VMEM((_NCH, _CH), jnp.int32),
            pltpu.VMEM((_BPW,), jnp.int32),
            pltpu.VMEM((_BPW,), jnp.int32),
            pltpu.VMEM((_CH, 128), jnp.float32),
            pltpu.VMEM((_CH, 128), jnp.float32),
            pltpu.VMEM((_CH, 128), jnp.float32),
            pltpu.VMEM((_CH, 128), jnp.float32),
            pltpu.VMEM((_BPW * _EMBED_DIM,), jnp.float32),
            pltpu.SemaphoreType.DMA,
            pltpu.SemaphoreType.DMA,
            pltpu.SemaphoreType.DMA,
            pltpu.SemaphoreType.DMA,
        ],
    )
    def dh_embed(feat_hbm, tbl_hbm, out_hbm,
                 feat_v, g1_v, g2_v, s1_v, s2_v,
                 rows1a_v, rows1b_v, rows2a_v, rows2b_v, out_v,
                 sem10, sem11, sem20, sem21):
        wid = lax.axis_index("s") * _NC + lax.axis_index("c")
        base = wid * _BPW
        pltpu.sync_copy(feat_hbm.at[pl.ds(base, _BPW)], feat_v)

        for c in range(_NCH):
            def hash_body(j, _, c=c):
                g = c * (_CH // _L) + j
                x = feat_v[pl.ds(g * _L, _L)].astype(jnp.uint32)
                h1 = (x * jnp.uint32(2654435761)) % jnp.uint32(_NUM_BUCKETS)
                y = x ^ (x >> jnp.uint32(16))
                y = y * jnp.uint32(0x45D9F3B)
                y = y ^ (y >> jnp.uint32(13))
                h2 = y % jnp.uint32(_NUM_BUCKETS)
                h1 = h1.astype(jnp.int32)
                h2 = h2.astype(jnp.int32)
                g1_v[c, pl.ds(j * _L, _L)] = h1 >> 3
                g2_v[c, pl.ds(j * _L, _L)] = h2 >> 3
                s1_v[pl.ds(g * _L, _L)] = (h1 & 7) << 4
                s2_v[pl.ds(g * _L, _L)] = (h2 & 7) << 4
                return 0

            lax.fori_loop(0, _CH // _L, hash_body, 0)

        bufs = ((rows1a_v, rows2a_v, sem10, sem20),
                (rows1b_v, rows2b_v, sem11, sem21))

        def start(c):
            b1, b2, s1, s2 = bufs[c % 2]
            return (
                pltpu.async_copy(tbl_hbm.at[g1_v.at[c]], b1, s1),
                pltpu.async_copy(tbl_hbm.at[g2_v.at[c]], b2, s2),
            )

        pend = start(0)
        for c in range(_NCH):
            nxt = start(c + 1) if c + 1 < _NCH else None
            pend[0].wait()
            pend[1].wait()
            r1, r2 = bufs[c % 2][0], bufs[c % 2][1]

            def extract(j, _, c=c, r1=r1, r2=r2):
                sv1 = s1_v[pl.ds((c * (_CH // _L) + j) * _L, _L)]
                sv2 = s2_v[pl.ds((c * (_CH // _L) + j) * _L, _L)]
                for l in range(_L):
                    i = j * _L + l
                    v1 = r1[i, pl.ds(sv1[l], _L)]
                    v2 = r2[i, pl.ds(sv2[l], _L)]
                    out_v[pl.ds((c * _CH + i) * _EMBED_DIM, _L)] = v1 + v2
                return 0

            lax.fori_loop(0, _CH // _L, extract, 0)
            pend = nxt

        pltpu.sync_copy(out_v, out_hbm.at[pl.ds(base * _EMBED_DIM,
                                                _BPW * _EMBED_DIM)])

    return dh_embed


_dh_embed = _make_sc_kernel()


def kernel(features, table):
    tbl_lin = _detile(table.T)
    out = _dh_embed(features, tbl_lin)
    return out.reshape(_BATCH, _EMBED_DIM)


# speedup vs baseline: 5.1115x; 3.2842x over previous
import jax
import jax.numpy as jnp
from jax import lax
from jax.experimental import pallas as pl
from jax.experimental.pallas import tpu as pltpu
from jax.experimental.pallas import tpu_sc as plsc

_NUM_BUCKETS = 1000000
_EMBED_DIM = 16
_BATCH = 16384
_NC = 2
_NS = 16
_L = 16
_NW = _NC * _NS
_BPW = _BATCH // _NW
_CH = 128
_NCH = _BPW // _CH
_GROUPS = _NUM_BUCKETS // 8

_DT_C = 32768
_DT_R = _DT_C // 8
_DT_STEPS = -(-_NUM_BUCKETS // _DT_C)
_GROUPS_PAD = _DT_STEPS * _DT_R


def _detile_body(x_ref, y_ref):
    x = x_ref[...]
    stacked = jnp.concatenate(
        [x[:, m * _DT_R:(m + 1) * _DT_R] for m in range(8)], axis=0)
    y_ref[...] = jax.lax.dot_general(
        stacked, jnp.eye(128, dtype=jnp.float32),
        (((0,), (0,)), ((), ())),
        preferred_element_type=jnp.float32)


def _detile(t2):
    return pl.pallas_call(
        _detile_body,
        grid=(_DT_STEPS,),
        in_specs=[pl.BlockSpec((_EMBED_DIM, _DT_C), lambda i: (0, i))],
        out_specs=pl.BlockSpec((_DT_R, 128), lambda i: (i, 0)),
        out_shape=jax.ShapeDtypeStruct((_GROUPS_PAD, 128), jnp.float32),
    )(t2)


def _make_sc_kernel():
    mesh = plsc.VectorSubcoreMesh(core_axis_name="c", subcore_axis_name="s")

    @pl.kernel(
        mesh=mesh,
        out_type=jax.ShapeDtypeStruct((_BATCH * _EMBED_DIM,), jnp.float32),
        scratch_types=[
            pltpu.VMEM((_BPW,), jnp.int32),
            pltpu.VMEM((_NCH, _CH), jnp.int32),
            pltpu.VMEM((_NCH, _CH), jnp.int32),
            pltpu.VMEM((_BPW,), jnp.int32),
            pltpu.VMEM((_BPW,), jnp.int32),
            pltpu.VMEM((_CH, 128), jnp.float32),
            pltpu.VMEM((_CH, 128), jnp.float32),
            pltpu.VMEM((_CH, 128), jnp.float32),
            pltpu.VMEM((_CH, 128), jnp.float32),
            pltpu.VMEM((_BPW * _EMBED_DIM,), jnp.float32),
            pltpu.SemaphoreType.DMA,
            pltpu.SemaphoreType.DMA,
            pltpu.SemaphoreType.DMA,
            pltpu.SemaphoreType.DMA,
        ],
    )
    def dh_embed(feat_hbm, tbl_hbm, out_hbm,
                 feat_v, g1_v, g2_v, s1_v, s2_v,
                 rows1a_v, rows1b_v, rows2a_v, rows2b_v, out_v,
                 sem10, sem11, sem20, sem21):
        wid = lax.axis_index("s") * _NC + lax.axis_index("c")
        base = wid * _BPW
        pltpu.sync_copy(feat_hbm.at[pl.ds(base, _BPW)], feat_v)

        for c in range(_NCH):
            def hash_body(j, _, c=c):
                g = c * (_CH // _L) + j
                x = feat_v[pl.ds(g * _L, _L)].astype(jnp.uint32)
                h1 = (x * jnp.uint32(2654435761)) % jnp.uint32(_NUM_BUCKETS)
                y = x ^ (x >> jnp.uint32(16))
                y = y * jnp.uint32(0x45D9F3B)
                y = y ^ (y >> jnp.uint32(13))
                h2 = y % jnp.uint32(_NUM_BUCKETS)
                h1 = h1.astype(jnp.int32)
                h2 = h2.astype(jnp.int32)
                g1_v[c, pl.ds(j * _L, _L)] = ((h1 >> 15) << 12) | (h1 & 4095)
                g2_v[c, pl.ds(j * _L, _L)] = ((h2 >> 15) << 12) | (h2 & 4095)
                s1_v[pl.ds(g * _L, _L)] = ((h1 >> 12) & 7) << 4
                s2_v[pl.ds(g * _L, _L)] = ((h2 >> 12) & 7) << 4
                return 0

            lax.fori_loop(0, _CH // _L, hash_body, 0)

        bufs = ((rows1a_v, rows2a_v, sem10, sem20),
                (rows1b_v, rows2b_v, sem11, sem21))

        def start(c):
            b1, b2, s1, s2 = bufs[c % 2]
            return (
                pltpu.async_copy(tbl_hbm.at[g1_v.at[c]], b1, s1),
                pltpu.async_copy(tbl_hbm.at[g2_v.at[c]], b2, s2),
            )

        pend = start(0)
        for c in range(_NCH):
            nxt = start(c + 1) if c + 1 < _NCH else None
            pend[0].wait()
            pend[1].wait()
            r1, r2 = bufs[c % 2][0], bufs[c % 2][1]

            def extract(j, _, c=c, r1=r1, r2=r2):
                sv1 = s1_v[pl.ds((c * (_CH // _L) + j) * _L, _L)]
                sv2 = s2_v[pl.ds((c * (_CH // _L) + j) * _L, _L)]
                for l in range(_L):
                    i = j * _L + l
                    v1 = r1[i, pl.ds(sv1[l], _L)]
                    v2 = r2[i, pl.ds(sv2[l], _L)]
                    out_v[pl.ds((c * _CH + i) * _EMBED_DIM, _L)] = v1 + v2
                return 0

            lax.fori_loop(0, _CH // _L, extract, 0)
            pend = nxt

        pltpu.sync_copy(out_v, out_hbm.at[pl.ds(base * _EMBED_DIM,
                                                _BPW * _EMBED_DIM)])

    return dh_embed


_dh_embed = _make_sc_kernel()


def kernel(features, table):
    tbl_lin = _detile(table.T)
    out = _dh_embed(features, tbl_lin)
    return out.reshape(_BATCH, _EMBED_DIM)


# speedup vs baseline: 5.1561x; 1.0087x over previous
import jax
import jax.numpy as jnp
from jax import lax
from jax.experimental import pallas as pl
from jax.experimental.pallas import tpu as pltpu
from jax.experimental.pallas import tpu_sc as plsc

_NUM_BUCKETS = 1000000
_EMBED_DIM = 16
_BATCH = 16384
_NC = 2
_NS = 16
_L = 16
_NW = _NC * _NS
_BPW = _BATCH // _NW
_CH = 128
_NCH = _BPW // _CH
_GROUPS = _NUM_BUCKETS // 8

_DT_C = 32768
_DT_R = _DT_C // 8
_DT_STEPS = -(-_NUM_BUCKETS // _DT_C)
_GROUPS_PAD = _DT_STEPS * _DT_R


def _detile_body(x_ref, y_ref):
    x = x_ref[...]
    stacked = jnp.concatenate(
        [x[:, m * _DT_R:(m + 1) * _DT_R] for m in range(8)], axis=0)
    y_ref[...] = jax.lax.dot_general(
        stacked, jnp.eye(128, dtype=jnp.float32),
        (((0,), (0,)), ((), ())),
        preferred_element_type=jnp.float32)


def _detile(t2):
    return pl.pallas_call(
        _detile_body,
        grid=(_DT_STEPS,),
        in_specs=[pl.BlockSpec((_EMBED_DIM, _DT_C), lambda i: (0, i))],
        out_specs=pl.BlockSpec((_DT_R, 128), lambda i: (i, 0)),
        out_shape=jax.ShapeDtypeStruct((_GROUPS_PAD, 128), jnp.float32),
    )(t2)


def _out_t_body(x_ref, y_ref):
    y_ref[...] = jax.lax.dot_general(
        jnp.eye(_EMBED_DIM, dtype=jnp.float32), x_ref[...],
        (((1,), (1,)), ((), ())),
        preferred_element_type=jnp.float32)


def _out_transpose(out2d):
    return pl.pallas_call(
        _out_t_body,
        out_shape=jax.ShapeDtypeStruct((_EMBED_DIM, _BATCH), jnp.float32),
    )(out2d)


def _make_sc_kernel():
    mesh = plsc.VectorSubcoreMesh(core_axis_name="c", subcore_axis_name="s")

    @pl.kernel(
        mesh=mesh,
        out_type=jax.ShapeDtypeStruct((_BATCH * _EMBED_DIM,), jnp.float32),
        scratch_types=[
            pltpu.VMEM((_BPW,), jnp.int32),
            pltpu.VMEM((_NCH, _CH), jnp.int32),
            pltpu.VMEM((_NCH, _CH), jnp.int32),
            pltpu.VMEM((_BPW,), jnp.int32),
            pltpu.VMEM((_BPW,), jnp.int32),
            pltpu.VMEM((_CH, 128), jnp.float32),
            pltpu.VMEM((_CH, 128), jnp.float32),
            pltpu.VMEM((_CH, 128), jnp.float32),
            pltpu.VMEM((_CH, 128), jnp.float32),
            pltpu.VMEM((_BPW * _EMBED_DIM,), jnp.float32),
            pltpu.SemaphoreType.DMA,
            pltpu.SemaphoreType.DMA,
            pltpu.SemaphoreType.DMA,
            pltpu.SemaphoreType.DMA,
        ],
    )
    def dh_embed(feat_hbm, tbl_hbm, out_hbm,
                 feat_v, g1_v, g2_v, s1_v, s2_v,
                 rows1a_v, rows1b_v, rows2a_v, rows2b_v, out_v,
                 sem10, sem11, sem20, sem21):
        wid = lax.axis_index("s") * _NC + lax.axis_index("c")
        base = wid * _BPW
        pltpu.sync_copy(feat_hbm.at[pl.ds(base, _BPW)], feat_v)

        for c in range(_NCH):
            def hash_body(j, _, c=c):
                g = c * (_CH // _L) + j
                x = feat_v[pl.ds(g * _L, _L)].astype(jnp.uint32)
                h1 = (x * jnp.uint32(2654435761)) % jnp.uint32(_NUM_BUCKETS)
                y = x ^ (x >> jnp.uint32(16))
                y = y * jnp.uint32(0x45D9F3B)
                y = y ^ (y >> jnp.uint32(13))
                h2 = y % jnp.uint32(_NUM_BUCKETS)
                h1 = h1.astype(jnp.int32)
                h2 = h2.astype(jnp.int32)
                g1_v[c, pl.ds(j * _L, _L)] = ((h1 >> 15) << 12) | (h1 & 4095)
                g2_v[c, pl.ds(j * _L, _L)] = ((h2 >> 15) << 12) | (h2 & 4095)
                s1_v[pl.ds(g * _L, _L)] = ((h1 >> 12) & 7) << 4
                s2_v[pl.ds(g * _L, _L)] = ((h2 >> 12) & 7) << 4
                return 0

            lax.fori_loop(0, _CH // _L, hash_body, 0)

        bufs = ((rows1a_v, rows2a_v, sem10, sem20),
                (rows1b_v, rows2b_v, sem11, sem21))

        def start(c):
            b1, b2, s1, s2 = bufs[c % 2]
            return (
                pltpu.async_copy(tbl_hbm.at[g1_v.at[c]], b1, s1),
                pltpu.async_copy(tbl_hbm.at[g2_v.at[c]], b2, s2),
            )

        pend = start(0)
        for c in range(_NCH):
            nxt = start(c + 1) if c + 1 < _NCH else None
            pend[0].wait()
            pend[1].wait()
            r1, r2 = bufs[c % 2][0], bufs[c % 2][1]

            def extract(j, _, c=c, r1=r1, r2=r2):
                sv1 = s1_v[pl.ds((c * (_CH // _L) + j) * _L, _L)]
                sv2 = s2_v[pl.ds((c * (_CH // _L) + j) * _L, _L)]
                for l in range(_L):
                    i = j * _L + l
                    v1 = r1[i, pl.ds(sv1[l], _L)]
                    v2 = r2[i, pl.ds(sv2[l], _L)]
                    out_v[pl.ds((c * _CH + i) * _EMBED_DIM, _L)] = v1 + v2
                return 0

            lax.fori_loop(0, _CH // _L, extract, 0)
            pend = nxt

        pltpu.sync_copy(out_v, out_hbm.at[pl.ds(base * _EMBED_DIM,
                                                _BPW * _EMBED_DIM)])

    return dh_embed


_dh_embed = _make_sc_kernel()


def kernel(features, table):
    tbl_lin = _detile(table.T)
    out = _dh_embed(features, tbl_lin)
    return _out_transpose(out.reshape(_BATCH, _EMBED_DIM)).T


# speedup vs baseline: 5.5913x; 1.0844x over previous
import jax
import jax.numpy as jnp
from jax import lax
from jax.experimental import pallas as pl
from jax.experimental.pallas import tpu as pltpu
from jax.experimental.pallas import tpu_sc as plsc

_NUM_BUCKETS = 1000000
_EMBED_DIM = 16
_BATCH = 16384
_NC = 2
_NS = 16
_L = 16
_NW = _NC * _NS
_BPW = _BATCH // _NW
_CH = 128
_NCH = _BPW // _CH
_GROUPS = _NUM_BUCKETS // 8

_DT_C = 65536
_DT_R = _DT_C // 8
_DT_STEPS = -(-_NUM_BUCKETS // _DT_C)
_GROUPS_PAD = _DT_STEPS * _DT_R


def _detile_body(x_ref, y_ref):
    x = x_ref[...]
    stacked = jnp.concatenate(
        [x[:, m * _DT_R:(m + 1) * _DT_R] for m in range(8)], axis=0)
    y_ref[...] = jax.lax.dot_general(
        stacked, jnp.eye(128, dtype=jnp.float32),
        (((0,), (0,)), ((), ())),
        preferred_element_type=jnp.float32)


def _detile(t2):
    return pl.pallas_call(
        _detile_body,
        grid=(_DT_STEPS,),
        in_specs=[pl.BlockSpec((_EMBED_DIM, _DT_C), lambda i: (0, i))],
        out_specs=pl.BlockSpec((_DT_R, 128), lambda i: (i, 0)),
        out_shape=jax.ShapeDtypeStruct((_GROUPS_PAD, 128), jnp.float32),
    )(t2)


def _out_t_body(x_ref, y_ref):
    y_ref[...] = jax.lax.dot_general(
        jnp.eye(_EMBED_DIM, dtype=jnp.float32), x_ref[...],
        (((1,), (1,)), ((), ())),
        preferred_element_type=jnp.float32)


def _out_transpose(out2d):
    return pl.pallas_call(
        _out_t_body,
        out_shape=jax.ShapeDtypeStruct((_EMBED_DIM, _BATCH), jnp.float32),
    )(out2d)


def _make_sc_kernel():
    mesh = plsc.VectorSubcoreMesh(core_axis_name="c", subcore_axis_name="s")

    @pl.kernel(
        mesh=mesh,
        out_type=jax.ShapeDtypeStruct((_BATCH * _EMBED_DIM,), jnp.float32),
        scratch_types=[
            pltpu.VMEM((_BPW,), jnp.int32),
            pltpu.VMEM((_NCH, _CH), jnp.int32),
            pltpu.VMEM((_NCH, _CH), jnp.int32),
            pltpu.VMEM((_BPW,), jnp.int32),
            pltpu.VMEM((_BPW,), jnp.int32),
            pltpu.VMEM((_CH, 128), jnp.float32),
            pltpu.VMEM((_CH, 128), jnp.float32),
            pltpu.VMEM((_CH, 128), jnp.float32),
            pltpu.VMEM((_CH, 128), jnp.float32),
            pltpu.VMEM((_BPW * _EMBED_DIM,), jnp.float32),
            pltpu.SemaphoreType.DMA,
            pltpu.SemaphoreType.DMA,
            pltpu.SemaphoreType.DMA,
            pltpu.SemaphoreType.DMA,
        ],
    )
    def dh_embed(feat_hbm, tbl_hbm, out_hbm,
                 feat_v, g1_v, g2_v, s1_v, s2_v,
                 rows1a_v, rows1b_v, rows2a_v, rows2b_v, out_v,
                 sem10, sem11, sem20, sem21):
        wid = lax.axis_index("s") * _NC + lax.axis_index("c")
        base = wid * _BPW
        pltpu.sync_copy(feat_hbm.at[pl.ds(base, _BPW)], feat_v)

        for c in range(_NCH):
            def hash_body(j, _, c=c):
                g = c * (_CH // _L) + j
                x = feat_v[pl.ds(g * _L, _L)].astype(jnp.uint32)
                h1 = (x * jnp.uint32(2654435761)) % jnp.uint32(_NUM_BUCKETS)
                y = x ^ (x >> jnp.uint32(16))
                y = y * jnp.uint32(0x45D9F3B)
                y = y ^ (y >> jnp.uint32(13))
                h2 = y % jnp.uint32(_NUM_BUCKETS)
                h1 = h1.astype(jnp.int32)
                h2 = h2.astype(jnp.int32)
                g1_v[c, pl.ds(j * _L, _L)] = ((h1 >> 16) << 13) | (h1 & 8191)
                g2_v[c, pl.ds(j * _L, _L)] = ((h2 >> 16) << 13) | (h2 & 8191)
                s1_v[pl.ds(g * _L, _L)] = ((h1 >> 13) & 7) << 4
                s2_v[pl.ds(g * _L, _L)] = ((h2 >> 13) & 7) << 4
                return 0

            lax.fori_loop(0, _CH // _L, hash_body, 0)

        bufs = ((rows1a_v, rows2a_v, sem10, sem20),
                (rows1b_v, rows2b_v, sem11, sem21))

        def start(c):
            b1, b2, s1, s2 = bufs[c % 2]
            return (
                pltpu.async_copy(tbl_hbm.at[g1_v.at[c]], b1, s1),
                pltpu.async_copy(tbl_hbm.at[g2_v.at[c]], b2, s2),
            )

        pend = start(0)
        for c in range(_NCH):
            nxt = start(c + 1) if c + 1 < _NCH else None
            pend[0].wait()
            pend[1].wait()
            r1, r2 = bufs[c % 2][0], bufs[c % 2][1]

            def extract(j, _, c=c, r1=r1, r2=r2):
                sv1 = s1_v[pl.ds((c * (_CH // _L) + j) * _L, _L)]
                sv2 = s2_v[pl.ds((c * (_CH // _L) + j) * _L, _L)]
                for l in range(_L):
                    i = j * _L + l
                    v1 = r1[i, pl.ds(sv1[l], _L)]
                    v2 = r2[i, pl.ds(sv2[l], _L)]
                    out_v[pl.ds((c * _CH + i) * _EMBED_DIM, _L)] = v1 + v2
                return 0

            lax.fori_loop(0, _CH // _L, extract, 0)
            pend = nxt

        pltpu.sync_copy(out_v, out_hbm.at[pl.ds(base * _EMBED_DIM,
                                                _BPW * _EMBED_DIM)])

    return dh_embed


_dh_embed = _make_sc_kernel()


def kernel(features, table):
    tbl_lin = _detile(table.T)
    out = _dh_embed(features, tbl_lin)
    return _out_transpose(out.reshape(_BATCH, _EMBED_DIM)).T


# speedup vs baseline: 5.6556x; 1.0115x over previous
import jax
import jax.numpy as jnp
from jax import lax
from jax.experimental import pallas as pl
from jax.experimental.pallas import tpu as pltpu
from jax.experimental.pallas import tpu_sc as plsc

_NUM_BUCKETS = 1000000
_EMBED_DIM = 16
_BATCH = 16384
_NC = 2
_NS = 16
_L = 16
_NW = _NC * _NS
_BPW = _BATCH // _NW
_CH = 128
_NCH = _BPW // _CH
_GROUPS = _NUM_BUCKETS // 8

_DT_C = 131072
_DT_R = _DT_C // 8
_DT_STEPS = -(-_NUM_BUCKETS // _DT_C)
_GROUPS_PAD = _DT_STEPS * _DT_R


def _detile_body(x_ref, y_ref):
    x = x_ref[...]
    stacked = jnp.concatenate(
        [x[:, m * _DT_R:(m + 1) * _DT_R] for m in range(8)], axis=0)
    y_ref[...] = jax.lax.dot_general(
        stacked, jnp.eye(128, dtype=jnp.float32),
        (((0,), (0,)), ((), ())),
        preferred_element_type=jnp.float32)


def _detile(t2):
    return pl.pallas_call(
        _detile_body,
        grid=(_DT_STEPS,),
        in_specs=[pl.BlockSpec((_EMBED_DIM, _DT_C), lambda i: (0, i))],
        out_specs=pl.BlockSpec((_DT_R, 128), lambda i: (i, 0)),
        out_shape=jax.ShapeDtypeStruct((_GROUPS_PAD, 128), jnp.float32),
    )(t2)


def _out_t_body(x_ref, y_ref):
    y_ref[...] = jax.lax.dot_general(
        jnp.eye(_EMBED_DIM, dtype=jnp.float32), x_ref[...],
        (((1,), (1,)), ((), ())),
        preferred_element_type=jnp.float32)


def _out_transpose(out2d):
    return pl.pallas_call(
        _out_t_body,
        out_shape=jax.ShapeDtypeStruct((_EMBED_DIM, _BATCH), jnp.float32),
    )(out2d)


def _make_sc_kernel():
    mesh = plsc.VectorSubcoreMesh(core_axis_name="c", subcore_axis_name="s")

    @pl.kernel(
        mesh=mesh,
        out_type=jax.ShapeDtypeStruct((_BATCH * _EMBED_DIM,), jnp.float32),
        scratch_types=[
            pltpu.VMEM((_BPW,), jnp.int32),
            pltpu.VMEM((_NCH, _CH), jnp.int32),
            pltpu.VMEM((_NCH, _CH), jnp.int32),
            pltpu.VMEM((_BPW,), jnp.int32),
            pltpu.VMEM((_BPW,), jnp.int32),
            pltpu.VMEM((_CH, 128), jnp.float32),
            pltpu.VMEM((_CH, 128), jnp.float32),
            pltpu.VMEM((_CH, 128), jnp.float32),
            pltpu.VMEM((_CH, 128), jnp.float32),
            pltpu.VMEM((_BPW * _EMBED_DIM,), jnp.float32),
            pltpu.SemaphoreType.DMA,
            pltpu.SemaphoreType.DMA,
            pltpu.SemaphoreType.DMA,
            pltpu.SemaphoreType.DMA,
        ],
    )
    def dh_embed(feat_hbm, tbl_hbm, out_hbm,
                 feat_v, g1_v, g2_v, s1_v, s2_v,
                 rows1a_v, rows1b_v, rows2a_v, rows2b_v, out_v,
                 sem10, sem11, sem20, sem21):
        wid = lax.axis_index("s") * _NC + lax.axis_index("c")
        base = wid * _BPW
        pltpu.sync_copy(feat_hbm.at[pl.ds(base, _BPW)], feat_v)

        for c in range(_NCH):
            def hash_body(j, _, c=c):
                g = c * (_CH // _L) + j
                x = feat_v[pl.ds(g * _L, _L)].astype(jnp.uint32)
                h1 = (x * jnp.uint32(2654435761)) % jnp.uint32(_NUM_BUCKETS)
                y = x ^ (x >> jnp.uint32(16))
                y = y * jnp.uint32(0x45D9F3B)
                y = y ^ (y >> jnp.uint32(13))
                h2 = y % jnp.uint32(_NUM_BUCKETS)
                h1 = h1.astype(jnp.int32)
                h2 = h2.astype(jnp.int32)
                g1_v[c, pl.ds(j * _L, _L)] = ((h1 >> 17) << 14) | (h1 & 16383)
                g2_v[c, pl.ds(j * _L, _L)] = ((h2 >> 17) << 14) | (h2 & 16383)
                s1_v[pl.ds(g * _L, _L)] = ((h1 >> 14) & 7) << 4
                s2_v[pl.ds(g * _L, _L)] = ((h2 >> 14) & 7) << 4
                return 0

            lax.fori_loop(0, _CH // _L, hash_body, 0)

        bufs = ((rows1a_v, rows2a_v, sem10, sem20),
                (rows1b_v, rows2b_v, sem11, sem21))

        def start(c):
            b1, b2, s1, s2 = bufs[c % 2]
            return (
                pltpu.async_copy(tbl_hbm.at[g1_v.at[c]], b1, s1),
                pltpu.async_copy(tbl_hbm.at[g2_v.at[c]], b2, s2),
            )

        pend = start(0)
        for c in range(_NCH):
            nxt = start(c + 1) if c + 1 < _NCH else None
            pend[0].wait()
            pend[1].wait()
            r1, r2 = bufs[c % 2][0], bufs[c % 2][1]

            def extract(j, _, c=c, r1=r1, r2=r2):
                sv1 = s1_v[pl.ds((c * (_CH // _L) + j) * _L, _L)]
                sv2 = s2_v[pl.ds((c * (_CH // _L) + j) * _L, _L)]
                for l in range(_L):
                    i = j * _L + l
                    v1 = r1[i, pl.ds(sv1[l], _L)]
                    v2 = r2[i, pl.ds(sv2[l], _L)]
                    out_v[pl.ds((c * _CH + i) * _EMBED_DIM, _L)] = v1 + v2
                return 0

            lax.fori_loop(0, _CH // _L, extract, 0)
            pend = nxt

        pltpu.sync_copy(out_v, out_hbm.at[pl.ds(base * _EMBED_DIM,
                                                _BPW * _EMBED_DIM)])

    return dh_embed


_dh_embed = _make_sc_kernel()


def kernel(features, table):
    tbl_lin = _detile(table.T)
    out = _dh_embed(features, tbl_lin)
    return _out_transpose(out.reshape(_BATCH, _EMBED_DIM)).T
